# baseline (speedup 1.0000x reference)
import functools

import jax
import jax.numpy as jnp
from jax import lax
from jax.experimental import pallas as pl
from jax.experimental.pallas import tpu as pltpu
from jax.experimental.pallas import tpu_sc as plsc

NC = 2
NS = 16
NW = NC * NS
L = 16
CHUNK = 128

_SIN_C = (-1.0 / 6.0, 1.0 / 120.0, -1.0 / 5040.0, 1.0 / 362880.0)
_COS_C = (-0.5, 1.0 / 24.0, -1.0 / 720.0, 1.0 / 40320.0, -1.0 / 3628800.0)


def _sincos_base(x):
    t = x * x
    s = _SIN_C[3]
    for c in (_SIN_C[2], _SIN_C[1], _SIN_C[0]):
        s = s * t + c
    s = x * (s * t + 1.0)
    c = _COS_C[4]
    for cc in (_COS_C[3], _COS_C[2], _COS_C[1], _COS_C[0]):
        c = c * t + cc
    c = c * t + 1.0
    return s, c


def _axis_coord(t, res):
    xp = t * jnp.float32(res - 1)
    x0 = xp.astype(jnp.int32)
    w = xp - x0.astype(jnp.float32)
    x1 = jnp.minimum(x0 + 1, res - 1)
    return x0, x1, w


def _tri_body(n, res, fdim, nfreq, p_hbm, tab_hbm, aabb_hbm, out_hbm,
              p_v, aabb_v, idx_v, w_v, rows_v, out_v,
              sem_g0, sem_g1, sem_o0, sem_o1):
    cid = lax.axis_index("c")
    sid = lax.axis_index("s")
    wid = sid * NC + cid
    per_w = n // NW
    base0 = wid * per_w
    nch = per_w // CHUNK

    pltpu.sync_copy(aabb_hbm, aabb_v)
    iota = lax.iota(jnp.int32, L)

    def splat_i(v):
        return jnp.full((L,), v, jnp.int32)

    ab = [[aabb_v[r, c, :] for c in range(3)] for r in range(2)]

    ncols = 3 * fdim + 3 + 6 * nfreq
    plane_sz = res * res
    gsem = (sem_g0, sem_g1)
    osem = (sem_o0, sem_o1)

    def cbase(ci):
        return base0 + ci * CHUNK

    def load_p(ci, pb):
        pltpu.sync_copy(p_hbm.at[pl.ds(cbase(ci) * 3, CHUNK * 3)],
                        p_v.at[pb])

    def phase1(ci, pb):
        def grp1(g, c1):
            rows = g * L + iota
            rows3 = rows * 3
            px = plsc.load_gather(p_v.at[pb], [rows3])
            py = plsc.load_gather(p_v.at[pb], [rows3 + 1])
            pz = plsc.load_gather(p_v.at[pb], [rows3 + 2])
            ts = []
            for c, pv in enumerate((px, py, pz)):
                t = (pv - ab[0][c]) / (ab[1][c] - ab[0][c])
                ts.append(jnp.minimum(jnp.maximum(t, 0.0), 1.0))
            tx, ty, tz = ts
            x0, x1, wx = _axis_coord(tx, res)
            y0, y1, wy = _axis_coord(ty, res)
            z0, z1, wz = _axis_coord(tz, res)

            taps = []
            for pi, (ga0, ga1, gb0, gb1) in enumerate((
                    (x0, x1, y0, y1),
                    (y0, y1, z0, z1),
                    (x0, x1, z0, z1))):
                off = pi * plane_sz
                taps += [
                    gb0 * res + ga0 + off,
                    gb0 * res + ga1 + off,
                    gb1 * res + ga0 + off,
                    gb1 * res + ga1 + off,
                ]
            for t, rid in enumerate(taps):
                idx_v[pb, pl.ds(t * CHUNK + g * L, L)] = rid
            for t, wgt in enumerate((wx, wy, wz)):
                w_v[pb, pl.ds(t * CHUNK + g * L, L)] = wgt

            rowsnc = rows * ncols

            def putcol(col, vec):
                plsc.store_scatter(out_v.at[pb], [rowsnc + col], vec)

            xg = 2.0 * tx - 1.0
            yg = 2.0 * ty - 1.0
            zg = 2.0 * tz - 1.0
            pe = 3 * fdim
            putcol(pe + 0, xg)
            putcol(pe + 1, yg)
            putcol(pe + 2, zg)
            scs = [_sincos_base(v) for v in (xg, yg, zg)]
            for i in range(nfreq):
                for c in range(3):
                    s, co = scs[c]
                    putcol(pe + 3 + 6 * i + c, s)
                    putcol(pe + 6 + 6 * i + c, co)
                    if i + 1 < nfreq:
                        scs[c] = (2.0 * s * co, 1.0 - 2.0 * s * s)
            return c1

        lax.fori_loop(0, CHUNK // L, grp1, 0)

    def fire(pb):
        pltpu.async_copy(tab_hbm.at[idx_v.at[pb]], rows_v.at[pb], gsem[pb])

    def wait_gathers(pb):
        pltpu.make_async_copy(tab_hbm.at[idx_v.at[pb]], rows_v.at[pb],
                              gsem[pb]).wait()

    def phase3(ci, pb):
        zero = jnp.zeros((L,), jnp.int32)

        def ptbody(b, c2):
            bs = zero + b
            wx = plsc.load_gather(w_v.at[pb], [bs])
            wy = plsc.load_gather(w_v.at[pb], [bs + CHUNK])
            wz = plsc.load_gather(w_v.at[pb], [bs + 2 * CHUNK])
            ux, uy, uz = 1.0 - wx, 1.0 - wy, 1.0 - wz
            for pi, (wa, ua, wb2, ub) in enumerate(
                    ((wx, ux, wy, uy), (wy, uy, wz, uz), (wx, ux, wz, uz))):
                w4 = (ua * ub, wa * ub, ua * wb2, wa * wb2)
                acc = w4[0] * rows_v[pb, 4 * pi * CHUNK + b, :]
                for t in range(1, 4):
                    acc = acc + w4[t] * rows_v[
                        pb, (4 * pi + t) * CHUNK + b, :]
                out_v[pb, pl.ds(b * ncols + pi * fdim, fdim)] = acc
            return c2

        lax.fori_loop(0, CHUNK, ptbody, 0)

    def write_out(ci, pb):
        pltpu.async_copy(out_v.at[pb],
                         out_hbm.at[pl.ds(cbase(ci) * ncols, CHUNK * ncols)],
                         osem[pb])

    def wait_out(ci, pb):
        pltpu.make_async_copy(out_v.at[pb],
                              out_hbm.at[pl.ds(cbase(ci) * ncols,
                                               CHUNK * ncols)],
                              osem[pb]).wait()

    load_p(0, 0)
    phase1(0, 0)
    fire(0)

    def pair_body(k, carry):
        c0 = 2 * k
        c1 = 2 * k + 1

        @pl.when(k > 0)
        def _():
            wait_out(c1 - 2, 1)

        load_p(c1, 1)
        phase1(c1, 1)
        fire(1)

        wait_gathers(0)
        phase3(c0, 0)
        write_out(c0, 0)

        @pl.when(k + 1 < nch // 2)
        def _():
            wait_out(c0, 0)
            load_p(c0 + 2, 0)
            phase1(c0 + 2, 0)
            fire(0)

        wait_gathers(1)
        phase3(c1, 1)
        write_out(c1, 1)
        return carry

    lax.fori_loop(0, nch // 2, pair_body, 0)
    wait_out(nch - 2, 0)
    wait_out(nch - 1, 1)


def kernel(p, triplane, aabb):
    n = p.shape[0]
    nplane, fdim, res, _ = triplane.shape
    nfreq = 6
    ncols = nplane * fdim + 3 + 6 * nfreq
    table = triplane.transpose(0, 2, 3, 1).reshape(nplane * res * res, fdim)
    aabb_b = jnp.tile(aabb[:, :, None], (1, 1, L))

    mesh = plsc.VectorSubcoreMesh(core_axis_name="c", subcore_axis_name="s",
                                  num_cores=NC, num_subcores=NS)
    scratch = [
        pltpu.VMEM((2, CHUNK * 3), jnp.float32),
        pltpu.VMEM((2, 3, L), jnp.float32),
        pltpu.VMEM((2, 12 * CHUNK), jnp.int32),
        pltpu.VMEM((2, 12 * CHUNK), jnp.float32),
        pltpu.VMEM((2, 12 * CHUNK, fdim), jnp.float32),
        pltpu.VMEM((2, CHUNK * ncols), jnp.float32),
        pltpu.SemaphoreType.DMA,
        pltpu.SemaphoreType.DMA,
        pltpu.SemaphoreType.DMA,
        pltpu.SemaphoreType.DMA,
    ]
    body = functools.partial(_tri_body, n, res, fdim, nfreq)
    run = pl.kernel(
        body,
        out_type=jax.ShapeDtypeStruct((n * ncols,), jnp.float32),
        mesh=mesh,
        scratch_types=scratch,
        compiler_params=pltpu.CompilerParams(
            needs_layout_passes=False, use_tc_tiling_on_sc=False),
    )
    return run(p.reshape(-1), table, aabb_b).reshape(n, ncols)


# speedup vs baseline: 1.0618x; 1.0618x over previous
import functools

import jax
import jax.numpy as jnp
from jax import lax
from jax.experimental import pallas as pl
from jax.experimental.pallas import tpu as pltpu
from jax.experimental.pallas import tpu_sc as plsc

NC = 2
NS = 16
NW = NC * NS
L = 16
CHUNK = 128

_SIN_C = (-1.0 / 6.0, 1.0 / 120.0, -1.0 / 5040.0, 1.0 / 362880.0)
_COS_C = (-0.5, 1.0 / 24.0, -1.0 / 720.0, 1.0 / 40320.0, -1.0 / 3628800.0)


def _sincos_base(x):
    t = x * x
    s = _SIN_C[3]
    for c in (_SIN_C[2], _SIN_C[1], _SIN_C[0]):
        s = s * t + c
    s = x * (s * t + 1.0)
    c = _COS_C[4]
    for cc in (_COS_C[3], _COS_C[2], _COS_C[1], _COS_C[0]):
        c = c * t + cc
    c = c * t + 1.0
    return s, c


def _axis_coord(t, res):
    xp = t * jnp.float32(res - 1)
    x0 = xp.astype(jnp.int32)
    w = xp - x0.astype(jnp.float32)
    x1 = jnp.minimum(x0 + 1, res - 1)
    return x0, x1, w


def _tri_body(n, res, fdim, nfreq, p_hbm, tab_hbm, aabb_hbm, out_hbm,
              p_v, aabb_v, idx_v, w_v, rows_v, out_v,
              sem_g0, sem_g1, sem_o0, sem_o1):
    cid = lax.axis_index("c")
    sid = lax.axis_index("s")
    wid = sid * NC + cid
    per_w = n // NW
    base0 = wid * per_w
    nch = per_w // CHUNK

    pltpu.sync_copy(aabb_hbm, aabb_v)
    iota = lax.iota(jnp.int32, L)

    def splat_i(v):
        return jnp.full((L,), v, jnp.int32)

    ab = [[aabb_v[r, c, :] for c in range(3)] for r in range(2)]

    ncols = 3 * fdim + 3 + 6 * nfreq
    plane_sz = res * res
    gsem = (sem_g0, sem_g1)
    osem = (sem_o0, sem_o1)

    def cbase(ci):
        return base0 + ci * CHUNK

    def load_p(ci, pb):
        pltpu.sync_copy(p_hbm.at[pl.ds(cbase(ci) * 3, CHUNK * 3)],
                        p_v.at[pb])

    def phase1(ci, pb):
        def grp1(g, c1):
            rows = g * L + iota
            rows3 = rows * 3
            px = plsc.load_gather(p_v.at[pb], [rows3])
            py = plsc.load_gather(p_v.at[pb], [rows3 + 1])
            pz = plsc.load_gather(p_v.at[pb], [rows3 + 2])
            ts = []
            for c, pv in enumerate((px, py, pz)):
                t = (pv - ab[0][c]) / (ab[1][c] - ab[0][c])
                ts.append(jnp.minimum(jnp.maximum(t, 0.0), 1.0))
            tx, ty, tz = ts
            x0, x1, wx = _axis_coord(tx, res)
            y0, y1, wy = _axis_coord(ty, res)
            z0, z1, wz = _axis_coord(tz, res)

            taps = []
            for pi, (ga0, ga1, gb0, gb1, wa, wb) in enumerate((
                    (x0, x1, y0, y1, wx, wy),
                    (y0, y1, z0, z1, wy, wz),
                    (x0, x1, z0, z1, wx, wz))):
                off = pi * plane_sz
                taps += [
                    (gb0 * res + ga0 + off, (1.0 - wa) * (1.0 - wb)),
                    (gb0 * res + ga1 + off, wa * (1.0 - wb)),
                    (gb1 * res + ga0 + off, (1.0 - wa) * wb),
                    (gb1 * res + ga1 + off, wa * wb),
                ]
            for t, (rid, wgt) in enumerate(taps):
                idx_v[pb, pl.ds(t * CHUNK + g * L, L)] = rid
                w_v[pb, pl.ds(t * CHUNK + g * L, L)] = wgt

            rowsnc = rows * ncols

            def putcol(col, vec):
                plsc.store_scatter(out_v.at[pb], [rowsnc + col], vec)

            xg = 2.0 * tx - 1.0
            yg = 2.0 * ty - 1.0
            zg = 2.0 * tz - 1.0
            pe = 3 * fdim
            putcol(pe + 0, xg)
            putcol(pe + 1, yg)
            putcol(pe + 2, zg)
            scs = [_sincos_base(v) for v in (xg, yg, zg)]
            for i in range(nfreq):
                for c in range(3):
                    s, co = scs[c]
                    putcol(pe + 3 + 6 * i + c, s)
                    putcol(pe + 6 + 6 * i + c, co)
                    if i + 1 < nfreq:
                        scs[c] = (2.0 * s * co, 1.0 - 2.0 * s * s)
            return c1

        lax.fori_loop(0, CHUNK // L, grp1, 0)

    def fire(pb):
        pltpu.async_copy(tab_hbm.at[idx_v.at[pb]], rows_v.at[pb], gsem[pb])

    def wait_gathers(pb):
        pltpu.make_async_copy(tab_hbm.at[idx_v.at[pb]], rows_v.at[pb],
                              gsem[pb]).wait()

    def phase3(ci, pb):
        zero = jnp.zeros((L,), jnp.int32)

        def ptbody(b, c2):
            bs = zero + b
            wb = [plsc.load_gather(w_v.at[pb], [bs + t * CHUNK])
                  for t in range(12)]
            rv = [rows_v[pb, t * CHUNK + b, :] for t in range(12)]
            for pi in range(3):
                t0 = 4 * pi
                acc = ((wb[t0] * rv[t0] + wb[t0 + 1] * rv[t0 + 1])
                       + (wb[t0 + 2] * rv[t0 + 2] + wb[t0 + 3] * rv[t0 + 3]))
                out_v[pb, pl.ds(b * ncols + pi * fdim, fdim)] = acc
            return c2

        lax.fori_loop(0, CHUNK, ptbody, 0)

    def write_out(ci, pb):
        pltpu.async_copy(out_v.at[pb],
                         out_hbm.at[pl.ds(cbase(ci) * ncols, CHUNK * ncols)],
                         osem[pb])

    def wait_out(ci, pb):
        pltpu.make_async_copy(out_v.at[pb],
                              out_hbm.at[pl.ds(cbase(ci) * ncols,
                                               CHUNK * ncols)],
                              osem[pb]).wait()

    load_p(0, 0)
    phase1(0, 0)
    fire(0)

    def pair_body(k, carry):
        c0 = 2 * k
        c1 = 2 * k + 1

        @pl.when(k > 0)
        def _():
            wait_out(c1 - 2, 1)

        load_p(c1, 1)
        phase1(c1, 1)
        fire(1)

        wait_gathers(0)
        phase3(c0, 0)
        write_out(c0, 0)

        @pl.when(k + 1 < nch // 2)
        def _():
            wait_out(c0, 0)
            load_p(c0 + 2, 0)
            phase1(c0 + 2, 0)
            fire(0)

        wait_gathers(1)
        phase3(c1, 1)
        write_out(c1, 1)
        return carry

    lax.fori_loop(0, nch // 2, pair_body, 0)
    wait_out(nch - 2, 0)
    wait_out(nch - 1, 1)


def kernel(p, triplane, aabb):
    n = p.shape[0]
    nplane, fdim, res, _ = triplane.shape
    nfreq = 6
    ncols = nplane * fdim + 3 + 6 * nfreq
    table = triplane.transpose(0, 2, 3, 1).reshape(nplane * res * res, fdim)
    aabb_b = jnp.tile(aabb[:, :, None], (1, 1, L))

    mesh = plsc.VectorSubcoreMesh(core_axis_name="c", subcore_axis_name="s",
                                  num_cores=NC, num_subcores=NS)
    scratch = [
        pltpu.VMEM((2, CHUNK * 3), jnp.float32),
        pltpu.VMEM((2, 3, L), jnp.float32),
        pltpu.VMEM((2, 12 * CHUNK), jnp.int32),
        pltpu.VMEM((2, 12 * CHUNK), jnp.float32),
        pltpu.VMEM((2, 12 * CHUNK, fdim), jnp.float32),
        pltpu.VMEM((2, CHUNK * ncols), jnp.float32),
        pltpu.SemaphoreType.DMA,
        pltpu.SemaphoreType.DMA,
        pltpu.SemaphoreType.DMA,
        pltpu.SemaphoreType.DMA,
    ]
    body = functools.partial(_tri_body, n, res, fdim, nfreq)
    run = pl.kernel(
        body,
        out_type=jax.ShapeDtypeStruct((n * ncols,), jnp.float32),
        mesh=mesh,
        scratch_types=scratch,
        compiler_params=pltpu.CompilerParams(
            needs_layout_passes=False, use_tc_tiling_on_sc=False),
    )
    return run(p.reshape(-1), table, aabb_b).reshape(n, ncols)


# speedup vs baseline: 1.0646x; 1.0027x over previous
import functools

import jax
import jax.numpy as jnp
from jax import lax
from jax.experimental import pallas as pl
from jax.experimental.pallas import tpu as pltpu
from jax.experimental.pallas import tpu_sc as plsc

NC = 2
NS = 16
NW = NC * NS
L = 16
CHUNK = 128

_SIN_C = (-1.0 / 6.0, 1.0 / 120.0, -1.0 / 5040.0, 1.0 / 362880.0)
_COS_C = (-0.5, 1.0 / 24.0, -1.0 / 720.0, 1.0 / 40320.0, -1.0 / 3628800.0)


def _sincos_base(x):
    t = x * x
    s = _SIN_C[3]
    for c in (_SIN_C[2], _SIN_C[1], _SIN_C[0]):
        s = s * t + c
    s = x * (s * t + 1.0)
    c = _COS_C[4]
    for cc in (_COS_C[3], _COS_C[2], _COS_C[1], _COS_C[0]):
        c = c * t + cc
    c = c * t + 1.0
    return s, c


def _axis_coord(t, res):
    xp = t * jnp.float32(res - 1)
    x0 = xp.astype(jnp.int32)
    w = xp - x0.astype(jnp.float32)
    x1 = jnp.minimum(x0 + 1, res - 1)
    return x0, x1, w


def _tri_body(n, res, fdim, nfreq, p_hbm, tab_hbm, aabb_hbm, out_hbm,
              p_v, aabb_v, idx_v, w_v, rows_v, out_v,
              sem_g0, sem_g1, sem_o0, sem_o1):
    cid = lax.axis_index("c")
    sid = lax.axis_index("s")
    wid = sid * NC + cid
    per_w = n // NW
    base0 = wid * per_w
    nch = per_w // CHUNK

    pltpu.sync_copy(aabb_hbm, aabb_v)
    iota = lax.iota(jnp.int32, L)

    def splat_i(v):
        return jnp.full((L,), v, jnp.int32)

    ab = [[aabb_v[r, c, :] for c in range(3)] for r in range(2)]

    ncols = 3 * fdim + 3 + 6 * nfreq
    plane_sz = res * res
    gsem = (sem_g0, sem_g1)
    osem = (sem_o0, sem_o1)

    def cbase(ci):
        return base0 + ci * CHUNK

    def load_p(ci, pb):
        pltpu.sync_copy(p_hbm.at[pl.ds(cbase(ci) * 3, CHUNK * 3)],
                        p_v.at[pb])

    def phase1(ci, pb):
        def grp1(g, c1):
            rows = g * L + iota
            rows3 = rows * 3
            px = plsc.load_gather(p_v.at[pb], [rows3])
            py = plsc.load_gather(p_v.at[pb], [rows3 + 1])
            pz = plsc.load_gather(p_v.at[pb], [rows3 + 2])
            ts = []
            for c, pv in enumerate((px, py, pz)):
                t = (pv - ab[0][c]) / (ab[1][c] - ab[0][c])
                ts.append(jnp.minimum(jnp.maximum(t, 0.0), 1.0))
            tx, ty, tz = ts
            x0, x1, wx = _axis_coord(tx, res)
            y0, y1, wy = _axis_coord(ty, res)
            z0, z1, wz = _axis_coord(tz, res)

            taps = []
            for pi, (ga0, ga1, gb0, gb1) in enumerate((
                    (x0, x1, y0, y1),
                    (y0, y1, z0, z1),
                    (x0, x1, z0, z1))):
                off = pi * plane_sz
                taps += [
                    gb0 * res + ga0 + off,
                    gb0 * res + ga1 + off,
                    gb1 * res + ga0 + off,
                    gb1 * res + ga1 + off,
                ]
            for t, rid in enumerate(taps):
                idx_v[pb, pl.ds(t * CHUNK + g * L, L)] = rid
            for t, wgt in enumerate((wx, wy, wz)):
                w_v[pb, pl.ds(t * CHUNK + g * L, L)] = wgt

            rowsnc = rows * ncols

            def putcol(col, vec):
                plsc.store_scatter(out_v.at[pb], [rowsnc + col], vec)

            xg = 2.0 * tx - 1.0
            yg = 2.0 * ty - 1.0
            zg = 2.0 * tz - 1.0
            pe = 3 * fdim
            putcol(pe + 0, xg)
            putcol(pe + 1, yg)
            putcol(pe + 2, zg)
            scs = [_sincos_base(v) for v in (xg, yg, zg)]
            for i in range(nfreq):
                for c in range(3):
                    s, co = scs[c]
                    putcol(pe + 3 + 6 * i + c, s)
                    putcol(pe + 6 + 6 * i + c, co)
                    if i + 1 < nfreq:
                        scs[c] = (2.0 * s * co, 1.0 - 2.0 * s * s)
            return c1

        lax.fori_loop(0, CHUNK // L, grp1, 0)

    def fire(pb):
        pltpu.async_copy(tab_hbm.at[idx_v.at[pb]], rows_v.at[pb], gsem[pb])

    def wait_gathers(pb):
        pltpu.make_async_copy(tab_hbm.at[idx_v.at[pb]], rows_v.at[pb],
                              gsem[pb]).wait()

    def phase3(ci, pb):
        zero = jnp.zeros((L,), jnp.int32)

        def ptbody(b, c2):
            bs = zero + b
            wx = plsc.load_gather(w_v.at[pb], [bs])
            wy = plsc.load_gather(w_v.at[pb], [bs + CHUNK])
            wz = plsc.load_gather(w_v.at[pb], [bs + 2 * CHUNK])
            ux, uy, uz = 1.0 - wx, 1.0 - wy, 1.0 - wz
            rv = [rows_v[pb, t * CHUNK + b, :] for t in range(12)]
            for pi, (wa, ua, wb2, ub) in enumerate(
                    ((wx, ux, wy, uy), (wy, uy, wz, uz), (wx, ux, wz, uz))):
                t0 = 4 * pi
                acc = ((ua * ub) * rv[t0] + (wa * ub) * rv[t0 + 1]
                       + ((ua * wb2) * rv[t0 + 2] + (wa * wb2) * rv[t0 + 3]))
                out_v[pb, pl.ds(b * ncols + pi * fdim, fdim)] = acc
            return c2

        lax.fori_loop(0, CHUNK, ptbody, 0)

    def write_out(ci, pb):
        pltpu.async_copy(out_v.at[pb],
                         out_hbm.at[pl.ds(cbase(ci) * ncols, CHUNK * ncols)],
                         osem[pb])

    def wait_out(ci, pb):
        pltpu.make_async_copy(out_v.at[pb],
                              out_hbm.at[pl.ds(cbase(ci) * ncols,
                                               CHUNK * ncols)],
                              osem[pb]).wait()

    load_p(0, 0)
    phase1(0, 0)
    fire(0)

    def pair_body(k, carry):
        c0 = 2 * k
        c1 = 2 * k + 1

        @pl.when(k > 0)
        def _():
            wait_out(c1 - 2, 1)

        load_p(c1, 1)
        phase1(c1, 1)
        fire(1)

        wait_gathers(0)
        phase3(c0, 0)
        write_out(c0, 0)

        @pl.when(k + 1 < nch // 2)
        def _():
            wait_out(c0, 0)
            load_p(c0 + 2, 0)
            phase1(c0 + 2, 0)
            fire(0)

        wait_gathers(1)
        phase3(c1, 1)
        write_out(c1, 1)
        return carry

    lax.fori_loop(0, nch // 2, pair_body, 0)
    wait_out(nch - 2, 0)
    wait_out(nch - 1, 1)


def kernel(p, triplane, aabb):
    n = p.shape[0]
    nplane, fdim, res, _ = triplane.shape
    nfreq = 6
    ncols = nplane * fdim + 3 + 6 * nfreq
    table = triplane.transpose(0, 2, 3, 1).reshape(nplane * res * res, fdim)
    aabb_b = jnp.tile(aabb[:, :, None], (1, 1, L))

    mesh = plsc.VectorSubcoreMesh(core_axis_name="c", subcore_axis_name="s",
                                  num_cores=NC, num_subcores=NS)
    scratch = [
        pltpu.VMEM((2, CHUNK * 3), jnp.float32),
        pltpu.VMEM((2, 3, L), jnp.float32),
        pltpu.VMEM((2, 12 * CHUNK), jnp.int32),
        pltpu.VMEM((2, 12 * CHUNK), jnp.float32),
        pltpu.VMEM((2, 12 * CHUNK, fdim), jnp.float32),
        pltpu.VMEM((2, CHUNK * ncols), jnp.float32),
        pltpu.SemaphoreType.DMA,
        pltpu.SemaphoreType.DMA,
        pltpu.SemaphoreType.DMA,
        pltpu.SemaphoreType.DMA,
    ]
    body = functools.partial(_tri_body, n, res, fdim, nfreq)
    run = pl.kernel(
        body,
        out_type=jax.ShapeDtypeStruct((n * ncols,), jnp.float32),
        mesh=mesh,
        scratch_types=scratch,
        compiler_params=pltpu.CompilerParams(
            needs_layout_passes=False, use_tc_tiling_on_sc=False),
    )
    return run(p.reshape(-1), table, aabb_b).reshape(n, ncols)
